# baseline (speedup 1.0000x reference)
import functools

import jax
import jax.numpy as jnp
from jax import lax
from jax.experimental import pallas as pl
from jax.experimental.pallas import tpu as pltpu
from jax.experimental.pallas import tpu_sc as plsc

_NTBL = 1000000
_BATCH = 16384
_FIELDS = 26
_DIM = 32
_ROWS = _BATCH * _FIELDS
_NW = 32
_RPW = _ROWS // _NW
_CHUNK = 1664
_NCH = _RPW // _CHUNK


@functools.partial(
    pl.kernel,
    mesh=plsc.VectorSubcoreMesh(core_axis_name="c", subcore_axis_name="s"),
    out_type=jax.ShapeDtypeStruct((_ROWS, _DIM), jnp.float32),
    scratch_types=[
        pltpu.VMEM((_RPW,), jnp.int32),
        pltpu.VMEM((2, _CHUNK, _DIM), jnp.float32),
        pltpu.SemaphoreType.DMA,
        pltpu.SemaphoreType.DMA,
        pltpu.SemaphoreType.DMA,
        pltpu.SemaphoreType.DMA,
    ],
    compiler_params=pltpu.CompilerParams(use_tc_tiling_on_sc=False),
)
def _gather_kernel(table, idx, out, idx_v, rows_v, sem_g0, sem_g1, sem_o0, sem_o1):
    w = lax.axis_index("s") * 2 + lax.axis_index("c")
    base = pl.multiple_of(w * _RPW, 8)
    pltpu.sync_copy(idx.at[pl.ds(base, _RPW)], idx_v)
    sems_g = (sem_g0, sem_g1)
    sems_o = (sem_o0, sem_o1)

    def gather(c):
        b = c % 2
        return pltpu.make_async_copy(
            table.at[idx_v.at[pl.ds(c * _CHUNK, _CHUNK)]], rows_v.at[b], sems_g[b]
        )

    def outcp(c):
        b = c % 2
        return pltpu.make_async_copy(
            rows_v.at[b], out.at[pl.ds(base + c * _CHUNK, _CHUNK)], sems_o[b]
        )

    gather(0).start()
    for c in range(_NCH):
        if c + 1 < _NCH:
            if c - 1 >= 0:
                outcp(c - 1).wait()
            gather(c + 1).start()
        gather(c).wait()
        outcp(c).start()
    outcp(_NCH - 2).wait()
    outcp(_NCH - 1).wait()


_QBLK = 512
_NQ = (_NTBL + _QBLK - 1) // _QBLK


def _format_body(in_ref, o_ref):
    x3 = in_ref[...].T.reshape(128, 4, 32)
    for j in range(4):
        o_ref[:, 32 * j:32 * (j + 1)] = x3[:, j, :]


_format_tc = pl.pallas_call(
    _format_body,
    grid=(_NQ,),
    in_specs=[pl.BlockSpec((_DIM, _QBLK), lambda q: (0, q))],
    out_specs=pl.BlockSpec((_QBLK // 4, 128), lambda q: (q, 0)),
    out_shape=jax.ShapeDtypeStruct((_NTBL * _DIM // 128, 128), jnp.float32),
)


def kernel(X, table):
    idx = X.reshape(_ROWS)
    tbl_lin = _format_tc(table.T)
    out = _gather_kernel(tbl_lin.reshape(_NTBL, _DIM), idx)
    return out.reshape(_BATCH, _FIELDS, _DIM)


# speedup vs baseline: 1.9219x; 1.9219x over previous
import functools

import jax
import jax.numpy as jnp
from jax import lax
from jax.experimental import pallas as pl
from jax.experimental.pallas import tpu as pltpu
from jax.experimental.pallas import tpu_sc as plsc

_NTBL = 1000000
_BATCH = 16384
_FIELDS = 26
_DIM = 32
_NW = 32
_BPW = _BATCH // _NW
_BCH = 64
_NCH = _BPW // _BCH


@functools.partial(
    pl.kernel,
    mesh=plsc.VectorSubcoreMesh(core_axis_name="c", subcore_axis_name="s"),
    out_type=jax.ShapeDtypeStruct((_BATCH, _FIELDS, _DIM), jnp.float32),
    scratch_types=[
        pltpu.VMEM((_FIELDS, _BPW), jnp.int32),
        pltpu.VMEM((2, _FIELDS, _BCH, _DIM), jnp.float32),
        pltpu.SemaphoreType.DMA,
        pltpu.SemaphoreType.DMA,
        pltpu.SemaphoreType.DMA,
        pltpu.SemaphoreType.DMA,
    ],
    compiler_params=pltpu.CompilerParams(use_tc_tiling_on_sc=False),
)
def _gather_kernel(table, idxT, out, idx_v, rows_v, sem_g0, sem_g1, sem_o0, sem_o1):
    w = lax.axis_index("s") * 2 + lax.axis_index("c")
    bbase = w * _BPW
    pltpu.sync_copy(idxT.at[:, pl.ds(bbase, _BPW)], idx_v)
    sems_g = (sem_g0, sem_g1)
    sems_o = (sem_o0, sem_o1)

    def gathers(c):
        b = c % 2
        return [
            pltpu.make_async_copy(
                table.at[idx_v.at[f, pl.ds(c * _BCH, _BCH)]],
                rows_v.at[b, f],
                sems_g[b],
            )
            for f in range(_FIELDS)
        ]

    def outcps(c):
        b = c % 2
        return [
            pltpu.make_async_copy(
                rows_v.at[b, f],
                out.at[pl.ds(bbase + c * _BCH, _BCH), f],
                sems_o[b],
            )
            for f in range(_FIELDS)
        ]

    for d in gathers(0):
        d.start()
    for c in range(_NCH):
        if c + 1 < _NCH:
            if c - 1 >= 0:
                for d in outcps(c - 1):
                    d.wait()
            for d in gathers(c + 1):
                d.start()
        for d in gathers(c):
            d.wait()
        for d in outcps(c):
            d.start()
    for c in (_NCH - 2, _NCH - 1):
        for d in outcps(c):
            d.wait()


def kernel(X, table):
    return _gather_kernel(table, X.T)
